# baseline (speedup 1.0000x reference)
import functools

import jax
import jax.numpy as jnp
from jax import lax
from jax.experimental import pallas as pl
from jax.experimental.pallas import tpu as pltpu
from jax.experimental.pallas import tpu_sc as plsc

N = 10000
E = 320000
NC, NS, LANES = 2, 16, 16
NW = NC * NS
EW = E // NW
K = 80
NCH = EW // K
NP = 10240
RSUB = NP // NS

R = 1000
G = N // R


def _shuf(v, idx16):
    return lax.gather(
        v, idx16[:, None],
        lax.GatherDimensionNumbers(offset_dims=(), collapsed_slice_dims=(0,),
                                   start_index_map=(0,)),
        (1,), mode=lax.GatherScatterMode.PROMISE_IN_BOUNDS)


def _edge_body1(ta_buf, d_buf, u_buf):
    lane = lax.iota(jnp.int32, 16)
    half = lane >> 3
    lo = lane < 8
    def f(k, c):
        a = ta_buf[k, pl.ds(64, 16)]
        dd = d_buf[k, pl.ds(0, 16)]
        e = a + dd
        w = jnp.exp(jnp.maximum(e, 0.2 * e))
        for ci in range(4):
            wb = _shuf(w, half + 2 * ci)
            u_buf[k, pl.ds(ci * 16, 16)] = wb * ta_buf[k, pl.ds(ci * 16, 16)]
        u_buf[k, pl.ds(64, 16)] = jnp.where(lo, w, 0.0)
        return c
    return f


def _edge_body2(ta_buf, d_buf, u_buf):
    nine = jnp.full((16,), 9, jnp.int32)
    def f(k, c):
        t2 = ta_buf[k, pl.ds(32, 16)]
        sa = _shuf(t2, nine)
        e = sa + d_buf[k, pl.ds(0, 16)]
        w = jnp.exp(jnp.maximum(e, 0.2 * e))
        u_buf[k, pl.ds(0, 16)] = w * ta_buf[k, pl.ds(0, 16)]
        u_buf[k, pl.ds(16, 16)] = w * ta_buf[k, pl.ds(16, 16)]
        u_buf[k, pl.ds(32, 16)] = w * t2
        return c
    return f


def _sc_edges(ta, d, src3, dst3, init, ta_w, d_w, u_w, edge_body):
    mesh = plsc.VectorSubcoreMesh(core_axis_name="c", subcore_axis_name="s")

    @functools.partial(
        pl.kernel,
        out_type=jax.ShapeDtypeStruct((NC, NP, u_w), jnp.float32),
        mesh=mesh,
        scratch_types=[
            pltpu.VMEM((K,), jnp.int32),
            pltpu.VMEM((K,), jnp.int32),
            pltpu.VMEM((K,), jnp.int32),
            pltpu.VMEM((K,), jnp.int32),
            pltpu.VMEM((K,), jnp.int32),
            pltpu.VMEM((K,), jnp.int32),
            pltpu.VMEM((K, ta_w), jnp.float32),
            pltpu.VMEM((K, d_w), jnp.float32),
            pltpu.VMEM((K, ta_w), jnp.float32),
            pltpu.VMEM((K, d_w), jnp.float32),
            pltpu.VMEM((K, u_w), jnp.float32),
            pltpu.VMEM_SHARED((NP, u_w), jnp.float32),
            pltpu.SemaphoreType.DMA,
            pltpu.SemaphoreType.DMA,
            pltpu.SemaphoreType.DMA,
            pltpu.SemaphoreType.DMA,
        ],
        compiler_params=pltpu.CompilerParams(use_tc_tiling_on_sc=False),
    )
    def k(ta_hbm, d_hbm, src_hbm, dst_hbm, init_hbm, acc_hbm,
          siA, diA, dsA, siB, diB, dsB, taA, dA, taB, dB, u_buf, acc_sh,
          semA, semB, simA, simB):
        cid = lax.axis_index("c")
        sid = lax.axis_index("s")
        wid = sid * NC + cid
        r0 = sid * RSUB
        pltpu.sync_copy(init_hbm.at[cid, pl.ds(r0, RSUB)],
                        acc_sh.at[pl.ds(r0, RSUB)])
        plsc.subcore_barrier()

        def fetch_idx(j, si, di, sem):
            pltpu.async_copy(src_hbm.at[wid, j], si, sem)
            pltpu.async_copy(dst_hbm.at[wid, j], di, sem)

        def wait_idx(si, di, sem):
            pltpu.make_async_copy(src_hbm.at[wid, 0], si, sem).wait()
            pltpu.make_async_copy(dst_hbm.at[wid, 0], di, sem).wait()

        def issue(si, di, ta_buf, d_buf, sem):
            pltpu.async_copy(ta_hbm.at[si], ta_buf, sem)
            pltpu.async_copy(d_hbm.at[di], d_buf, sem)

        def drain(si, di, ta_buf, d_buf, sem):
            pltpu.make_async_copy(ta_hbm.at[si], ta_buf, sem).wait()
            pltpu.make_async_copy(d_hbm.at[di], d_buf, sem).wait()

        def copy_idx(di, ds_):
            for ci in range(K // 16):
                ds_[pl.ds(ci * 16, 16)] = di[pl.ds(ci * 16, 16)]

        def half(j, si, di, ds_, ta_buf, d_buf, sem, sim):
            drain(si, di, ta_buf, d_buf, sem)
            copy_idx(di, ds_)
            nj = j + 2

            @pl.when(nj < NCH)
            def _():
                fetch_idx(nj, si, di, sim)

            body = edge_body(ta_buf, d_buf, u_buf)
            plsc.parallel_loop(0, K, 1, unroll=2)(lambda k: body(k, 0))
            pltpu.sync_copy(u_buf, acc_sh.at[ds_], add=True)

            @pl.when(nj < NCH)
            def _():
                wait_idx(si, di, sim)
                issue(si, di, ta_buf, d_buf, sem)

        fetch_idx(0, siA, diA, simA)
        wait_idx(siA, diA, simA)
        issue(siA, diA, taA, dA, semA)
        fetch_idx(1, siB, diB, simB)
        wait_idx(siB, diB, simB)
        issue(siB, diB, taB, dB, semB)

        def pair(p, carry):
            j = 2 * p
            half(j, siA, diA, dsA, taA, dA, semA, simA)
            half(j + 1, siB, diB, dsB, taB, dB, semB, simB)
            return carry

        lax.fori_loop(0, (NCH - 1) // 2, pair, 0)
        half(NCH - 1, siA, diA, dsA, taA, dA, semA, simA)
        plsc.subcore_barrier()
        pltpu.sync_copy(acc_sh.at[pl.ds(r0, RSUB)],
                        acc_hbm.at[cid, pl.ds(r0, RSUB)])

    return k(ta, d, src3, dst3, init)


def _tc1(x, W1, As8, Ad8, Rep8):
    def body(x_r, w1_r, as_r, ad_r, rep_r, ta_r, d_r, init_r):
        h = jnp.dot(x_r[...], w1_r[...], preferred_element_type=jnp.float32)
        sa = jnp.dot(h, as_r[...], preferred_element_type=jnp.float32)
        da = jnp.dot(h, ad_r[...], preferred_element_type=jnp.float32)
        e = sa + da
        w8 = jnp.exp(jnp.maximum(e, 0.2 * e))
        w64 = jnp.dot(w8, rep_r[...], preferred_element_type=jnp.float32)
        z8 = jnp.zeros((R, 8), jnp.float32)
        ta_r[...] = jnp.concatenate([h, sa, z8], axis=1)
        d_r[...] = jnp.concatenate([da, z8], axis=1)
        init_r[...] = jnp.concatenate([w64 * h, w8, z8], axis=1)

    return pl.pallas_call(
        body,
        grid=(G,),
        in_specs=[pl.BlockSpec((R, 128), lambda i: (i, 0)),
                  pl.BlockSpec((128, 64), lambda i: (0, 0)),
                  pl.BlockSpec((64, 8), lambda i: (0, 0)),
                  pl.BlockSpec((64, 8), lambda i: (0, 0)),
                  pl.BlockSpec((8, 64), lambda i: (0, 0))],
        out_specs=[pl.BlockSpec((R, 80), lambda i: (i, 0)),
                   pl.BlockSpec((R, 16), lambda i: (i, 0)),
                   pl.BlockSpec((R, 80), lambda i: (i, 0))],
        out_shape=[jax.ShapeDtypeStruct((N, 80), jnp.float32),
                   jax.ShapeDtypeStruct((N, 16), jnp.float32),
                   jax.ShapeDtypeStruct((N, 80), jnp.float32)],
    )(x, W1, As8, Ad8, Rep8)


def _tc2(a0, a1, b1, W2, As2, Ad2, Rep8):
    def body(a0_r, a1_r, b1_r, w2_r, as2_r, ad2_r, rep_r, ta_r, d_r, init_r):
        acc = a0_r[...] + a1_r[...]
        den = jnp.dot(acc[:, 64:72], rep_r[...],
                      preferred_element_type=jnp.float32)
        out1 = acc[:, :64] / den + b1_r[...]
        hh = jnp.where(out1 > 0, out1, jnp.exp(jnp.minimum(out1, 0.0)) - 1.0)
        h2 = jnp.dot(hh, w2_r[...], preferred_element_type=jnp.float32)
        sa = jnp.dot(h2, as2_r[...], preferred_element_type=jnp.float32)
        da = jnp.dot(h2, ad2_r[...], preferred_element_type=jnp.float32)
        e = sa + da
        w = jnp.exp(jnp.maximum(e, 0.2 * e))
        col = lax.broadcasted_iota(jnp.int32, (R, 8), 1)
        mid = jnp.where(col == 0, 1.0,
                        jnp.where(col == 1, sa[:, 0:1], 0.0)).astype(jnp.float32)
        hm = jnp.concatenate([h2, mid], axis=1)
        ta_r[...] = hm
        d_r[...] = da
        init_r[...] = hm * jnp.concatenate([w, w, w], axis=1)

    return pl.pallas_call(
        body,
        grid=(G,),
        in_specs=[pl.BlockSpec((R, 80), lambda i: (i, 0)),
                  pl.BlockSpec((R, 80), lambda i: (i, 0)),
                  pl.BlockSpec((1, 64), lambda i: (0, 0)),
                  pl.BlockSpec((64, 40), lambda i: (0, 0)),
                  pl.BlockSpec((40, 16), lambda i: (0, 0)),
                  pl.BlockSpec((40, 16), lambda i: (0, 0)),
                  pl.BlockSpec((8, 64), lambda i: (0, 0))],
        out_specs=[pl.BlockSpec((R, 48), lambda i: (i, 0)),
                   pl.BlockSpec((R, 16), lambda i: (i, 0)),
                   pl.BlockSpec((R, 48), lambda i: (i, 0))],
        out_shape=[jax.ShapeDtypeStruct((N, 48), jnp.float32),
                   jax.ShapeDtypeStruct((N, 16), jnp.float32),
                   jax.ShapeDtypeStruct((N, 48), jnp.float32)],
    )(a0, a1, b1, W2, As2, Ad2, Rep8)


def _tc3(a0, a1, b2):
    def body(a0_r, a1_r, b2_r, o_r):
        acc = a0_r[...] + a1_r[...]
        o = acc[:, :40] / acc[:, 40:41] + b2_r[...]
        m = jnp.max(o, axis=1, keepdims=True)
        lse = m + jnp.log(jnp.sum(jnp.exp(o - m), axis=1, keepdims=True))
        o_r[...] = o - lse

    return pl.pallas_call(
        body,
        grid=(G,),
        in_specs=[pl.BlockSpec((R, 48), lambda i: (i, 0)),
                  pl.BlockSpec((R, 48), lambda i: (i, 0)),
                  pl.BlockSpec((1, 40), lambda i: (0, 0))],
        out_specs=pl.BlockSpec((R, 40), lambda i: (i, 0)),
        out_shape=jax.ShapeDtypeStruct((N, 40), jnp.float32),
    )(a0, a1, b2)


def kernel(x, edge_index, W1, att_src1, att_dst1, b1,
           W2, att_src2, att_dst2, b2):
    f32 = jnp.float32
    hd = jnp.arange(64) // 8
    sel = (hd[:, None] == jnp.arange(8)[None, :]).astype(f32)
    As8 = att_src1.reshape(64)[:, None] * sel
    Ad8 = att_dst1.reshape(64)[:, None] * sel
    Rep8 = sel.T

    ta1, d1, init0 = _tc1(x, W1, As8, Ad8, Rep8)
    init0 = jnp.pad(init0, ((0, NP - N), (0, 0)))
    init1 = jnp.stack([init0, jnp.zeros_like(init0)])

    src3 = edge_index[0].reshape(NW, NCH, K)
    dst3 = edge_index[1].reshape(NW, NCH, K)

    acc1 = _sc_edges(ta1, d1, src3, dst3, init1, 80, 16, 80, _edge_body1)

    As2 = jnp.broadcast_to(att_src2.reshape(40)[:, None], (40, 16)).astype(f32)
    Ad2 = jnp.broadcast_to(att_dst2.reshape(40)[:, None], (40, 16)).astype(f32)
    ta2, d2, init2_0 = _tc2(acc1[0, :N], acc1[1, :N], b1.reshape(1, 64),
                            W2, As2, Ad2, Rep8)
    init2_0 = jnp.pad(init2_0, ((0, NP - N), (0, 0)))
    init2 = jnp.stack([init2_0, jnp.zeros_like(init2_0)])

    acc2 = _sc_edges(ta2, d2, src3, dst3, init2, 48, 16, 48, _edge_body2)

    return _tc3(acc2[0, :N], acc2[1, :N], b2.reshape(1, 40))


# speedup vs baseline: 1.1509x; 1.1509x over previous
import functools

import jax
import jax.numpy as jnp
from jax import lax
from jax.experimental import pallas as pl
from jax.experimental.pallas import tpu as pltpu
from jax.experimental.pallas import tpu_sc as plsc

N = 10000
E = 320000
NC, NS, LANES = 2, 16, 16
NW = NC * NS
EW = E // NW
K = 80
NCH = EW // K
NP = 10240
RSUB = NP // NS

R = 1024
G = NP // R

NEG = -1.0e30


def _shuf(v, idx16):
    return lax.gather(
        v, idx16[:, None],
        lax.GatherDimensionNumbers(offset_dims=(), collapsed_slice_dims=(0,),
                                   start_index_map=(0,)),
        (1,), mode=lax.GatherScatterMode.PROMISE_IN_BOUNDS)


def _edge_body1(ta_buf, d_buf, u_buf):
    lane = lax.iota(jnp.int32, 16)
    half = lane >> 3

    def f(k, c):
        a = ta_buf[k, pl.ds(64, 16)]
        dd = d_buf[k, pl.ds(0, 16)]
        e = a + dd
        w = jnp.exp(jnp.maximum(e, 0.2 * e))
        for ci in range(4):
            wb = _shuf(w, half + 2 * ci)
            u_buf[k, pl.ds(ci * 16, 16)] = wb * ta_buf[k, pl.ds(ci * 16, 16)]
        u_buf[k, pl.ds(64, 16)] = w
        return c
    return f


def _edge_body2(ta_buf, d_buf, u_buf):
    nine = jnp.full((16,), 9, jnp.int32)

    def f(k, c):
        t2 = ta_buf[k, pl.ds(32, 16)]
        sa = _shuf(t2, nine)
        e = sa + d_buf[k, pl.ds(0, 16)]
        w = jnp.exp(jnp.maximum(e, 0.2 * e))
        u_buf[k, pl.ds(0, 16)] = w * ta_buf[k, pl.ds(0, 16)]
        u_buf[k, pl.ds(16, 16)] = w * ta_buf[k, pl.ds(16, 16)]
        u_buf[k, pl.ds(32, 16)] = w * t2
        return c
    return f


def _sc_edges(ta, d, edges, init, ta_w, d_w, u_w, edge_body):
    mesh = plsc.VectorSubcoreMesh(core_axis_name="c", subcore_axis_name="s")

    @functools.partial(
        pl.kernel,
        out_type=jax.ShapeDtypeStruct((NC, NP, u_w), jnp.float32),
        mesh=mesh,
        scratch_types=[
            pltpu.VMEM((K,), jnp.int32),
            pltpu.VMEM((K,), jnp.int32),
            pltpu.VMEM((K,), jnp.int32),
            pltpu.VMEM((K,), jnp.int32),
            pltpu.VMEM((K,), jnp.int32),
            pltpu.VMEM((K,), jnp.int32),
            pltpu.VMEM((K, ta_w), jnp.float32),
            pltpu.VMEM((K, d_w), jnp.float32),
            pltpu.VMEM((K, ta_w), jnp.float32),
            pltpu.VMEM((K, d_w), jnp.float32),
            pltpu.VMEM((K, u_w), jnp.float32),
            pltpu.VMEM_SHARED((NP, u_w), jnp.float32),
            pltpu.SemaphoreType.DMA,
            pltpu.SemaphoreType.DMA,
            pltpu.SemaphoreType.DMA,
            pltpu.SemaphoreType.DMA,
        ],
        compiler_params=pltpu.CompilerParams(use_tc_tiling_on_sc=False),
    )
    def k(ta_hbm, d_hbm, edge_hbm, init_hbm, acc_hbm,
          siA, diA, dsA, siB, diB, dsB, taA, dA, taB, dB, u_buf, acc_sh,
          semA, semB, simA, simB):
        cid = lax.axis_index("c")
        sid = lax.axis_index("s")
        wid = sid * NC + cid
        r0 = sid * RSUB
        e0 = wid * EW

        def fetch_idx(j, si, di, sem):
            pltpu.async_copy(edge_hbm.at[0, pl.ds(e0 + j * K, K)], si, sem)
            pltpu.async_copy(edge_hbm.at[1, pl.ds(e0 + j * K, K)], di, sem)

        def wait_idx(si, di, sem):
            pltpu.make_async_copy(edge_hbm.at[0, pl.ds(0, K)], si, sem).wait()
            pltpu.make_async_copy(edge_hbm.at[1, pl.ds(0, K)], di, sem).wait()

        def issue(si, di, ta_buf, d_buf, sem):
            pltpu.async_copy(ta_hbm.at[si], ta_buf, sem)
            pltpu.async_copy(d_hbm.at[di], d_buf, sem)

        def drain(si, di, ta_buf, d_buf, sem):
            pltpu.make_async_copy(ta_hbm.at[si], ta_buf, sem).wait()
            pltpu.make_async_copy(d_hbm.at[di], d_buf, sem).wait()

        def copy_idx(di, ds_):
            for ci in range(K // 16):
                ds_[pl.ds(ci * 16, 16)] = di[pl.ds(ci * 16, 16)]

        fetch_idx(0, siA, diA, simA)
        fetch_idx(1, siB, diB, simB)

        @pl.when(cid == 0)
        def _():
            pltpu.sync_copy(init_hbm.at[pl.ds(r0, RSUB)],
                            acc_sh.at[pl.ds(r0, RSUB)])

        @pl.when(cid == 1)
        def _():
            def zrow(r, c):
                for ci in range(u_w // 16):
                    u_buf[r, pl.ds(ci * 16, 16)] = jnp.zeros((16,), jnp.float32)
                return c
            lax.fori_loop(0, K, zrow, 0)

            def zcp(i, c):
                pltpu.sync_copy(u_buf, acc_sh.at[pl.ds(r0 + i * K, K)])
                return c
            lax.fori_loop(0, RSUB // K, zcp, 0)

        wait_idx(siA, diA, simA)
        issue(siA, diA, taA, dA, semA)
        wait_idx(siB, diB, simB)
        issue(siB, diB, taB, dB, semB)
        plsc.subcore_barrier()

        def half(j, si, di, ds_, ta_buf, d_buf, sem, sim):
            drain(si, di, ta_buf, d_buf, sem)
            copy_idx(di, ds_)
            nj = j + 2

            @pl.when(nj < NCH)
            def _():
                fetch_idx(nj, si, di, sim)

            body = edge_body(ta_buf, d_buf, u_buf)
            plsc.parallel_loop(0, K, 1, unroll=2)(lambda kk: body(kk, 0))
            pltpu.sync_copy(u_buf, acc_sh.at[ds_], add=True)

            @pl.when(nj < NCH)
            def _():
                wait_idx(si, di, sim)
                issue(si, di, ta_buf, d_buf, sem)

        def pair(p, carry):
            j = 2 * p
            half(j, siA, diA, dsA, taA, dA, semA, simA)
            half(j + 1, siB, diB, dsB, taB, dB, semB, simB)
            return carry

        lax.fori_loop(0, (NCH - 1) // 2, pair, 0)
        half(NCH - 1, siA, diA, dsA, taA, dA, semA, simA)
        plsc.subcore_barrier()
        pltpu.sync_copy(acc_sh.at[pl.ds(r0, RSUB)],
                        acc_hbm.at[cid, pl.ds(r0, RSUB)])

    return k(ta, d, edges, init)


def _tc1(x, W1, As8, Ad8, Rep8):
    def body(x_r, w1_r, as_r, ad_r, rep_r, ta_r, d_r, init_r):
        h = jnp.dot(x_r[...], w1_r[...], preferred_element_type=jnp.float32)
        sa = jnp.dot(h, as_r[...], preferred_element_type=jnp.float32)
        da = jnp.dot(h, ad_r[...], preferred_element_type=jnp.float32)
        e = sa + da
        w8 = jnp.exp(jnp.maximum(e, 0.2 * e))
        w64 = jnp.dot(w8, rep_r[...], preferred_element_type=jnp.float32)
        z8 = jnp.zeros((R, 8), jnp.float32)
        ta_r[...] = jnp.concatenate([h, sa, jnp.full((R, 8), NEG)], axis=1)
        d_r[...] = jnp.concatenate([da, z8], axis=1)
        init_r[...] = jnp.concatenate([w64 * h, w8, z8], axis=1)

    return pl.pallas_call(
        body,
        grid=(G,),
        in_specs=[pl.BlockSpec((R, 128), lambda i: (i, 0)),
                  pl.BlockSpec((128, 64), lambda i: (0, 0)),
                  pl.BlockSpec((64, 8), lambda i: (0, 0)),
                  pl.BlockSpec((64, 8), lambda i: (0, 0)),
                  pl.BlockSpec((8, 64), lambda i: (0, 0))],
        out_specs=[pl.BlockSpec((R, 80), lambda i: (i, 0)),
                   pl.BlockSpec((R, 16), lambda i: (i, 0)),
                   pl.BlockSpec((R, 80), lambda i: (i, 0))],
        out_shape=[jax.ShapeDtypeStruct((N, 80), jnp.float32),
                   jax.ShapeDtypeStruct((N, 16), jnp.float32),
                   jax.ShapeDtypeStruct((NP, 80), jnp.float32)],
    )(x, W1, As8, Ad8, Rep8)


def _tc2(acc, b1, W2, As2, Ad2, Rep8):
    def body(a0_r, a1_r, b1_r, w2_r, as2_r, ad2_r, rep_r, ta_r, d_r, init_r):
        acc_b = a0_r[0] + a1_r[0]
        den = jnp.dot(acc_b[:, 64:72], rep_r[...],
                      preferred_element_type=jnp.float32)
        out1 = acc_b[:, :64] / den + b1_r[...]
        hh = jnp.where(out1 > 0, out1, jnp.exp(jnp.minimum(out1, 0.0)) - 1.0)
        h2 = jnp.dot(hh, w2_r[...], preferred_element_type=jnp.float32)
        sa = jnp.dot(h2, as2_r[...], preferred_element_type=jnp.float32)
        da = jnp.dot(h2, ad2_r[...], preferred_element_type=jnp.float32)
        e = sa + da
        w = jnp.exp(jnp.maximum(e, 0.2 * e))
        col = lax.broadcasted_iota(jnp.int32, (R, 8), 1)
        mid = jnp.where(col == 0, 1.0,
                        jnp.where(col == 1, sa[:, 0:1], 0.0)).astype(jnp.float32)
        hm = jnp.concatenate([h2, mid], axis=1)
        ta_r[...] = hm
        d_r[...] = da
        init_r[...] = hm * jnp.concatenate([w, w, w], axis=1)

    return pl.pallas_call(
        body,
        grid=(G,),
        in_specs=[pl.BlockSpec((1, R, 80), lambda i: (0, i, 0)),
                  pl.BlockSpec((1, R, 80), lambda i: (1, i, 0)),
                  pl.BlockSpec((1, 64), lambda i: (0, 0)),
                  pl.BlockSpec((64, 40), lambda i: (0, 0)),
                  pl.BlockSpec((40, 16), lambda i: (0, 0)),
                  pl.BlockSpec((40, 16), lambda i: (0, 0)),
                  pl.BlockSpec((8, 64), lambda i: (0, 0))],
        out_specs=[pl.BlockSpec((R, 48), lambda i: (i, 0)),
                   pl.BlockSpec((R, 16), lambda i: (i, 0)),
                   pl.BlockSpec((R, 48), lambda i: (i, 0))],
        out_shape=[jax.ShapeDtypeStruct((N, 48), jnp.float32),
                   jax.ShapeDtypeStruct((N, 16), jnp.float32),
                   jax.ShapeDtypeStruct((NP, 48), jnp.float32)],
    )(acc, acc, b1, W2, As2, Ad2, Rep8)


def _tc3(acc, b2):
    def body(a0_r, a1_r, b2_r, o_r):
        acc_b = a0_r[0] + a1_r[0]
        o = acc_b[:, :40] / acc_b[:, 40:41] + b2_r[...]
        m = jnp.max(o, axis=1, keepdims=True)
        lse = m + jnp.log(jnp.sum(jnp.exp(o - m), axis=1, keepdims=True))
        o_r[...] = o - lse

    return pl.pallas_call(
        body,
        grid=(G,),
        in_specs=[pl.BlockSpec((1, R, 48), lambda i: (0, i, 0)),
                  pl.BlockSpec((1, R, 48), lambda i: (1, i, 0)),
                  pl.BlockSpec((1, 40), lambda i: (0, 0))],
        out_specs=pl.BlockSpec((R, 40), lambda i: (i, 0)),
        out_shape=jax.ShapeDtypeStruct((N, 40), jnp.float32),
    )(acc, acc, b2)


def kernel(x, edge_index, W1, att_src1, att_dst1, b1,
           W2, att_src2, att_dst2, b2):
    f32 = jnp.float32
    hd = jnp.arange(64) // 8
    sel = (hd[:, None] == jnp.arange(8)[None, :]).astype(f32)
    As8 = att_src1.reshape(64)[:, None] * sel
    Ad8 = att_dst1.reshape(64)[:, None] * sel
    Rep8 = sel.T

    ta1, d1, init1 = _tc1(x, W1, As8, Ad8, Rep8)
    acc1 = _sc_edges(ta1, d1, edge_index, init1, 80, 16, 80, _edge_body1)

    As2 = jnp.broadcast_to(att_src2.reshape(40)[:, None], (40, 16)).astype(f32)
    Ad2 = jnp.broadcast_to(att_dst2.reshape(40)[:, None], (40, 16)).astype(f32)
    ta2, d2, init2 = _tc2(acc1, b1.reshape(1, 64), W2, As2, Ad2, Rep8)

    acc2 = _sc_edges(ta2, d2, edge_index, init2, 48, 16, 48, _edge_body2)

    return _tc3(acc2, b2.reshape(1, 40))


# speedup vs baseline: 1.2007x; 1.0433x over previous
import functools

import jax
import jax.numpy as jnp
from jax import lax
from jax.experimental import pallas as pl
from jax.experimental.pallas import tpu as pltpu
from jax.experimental.pallas import tpu_sc as plsc

N = 10000
E = 320000
NC, NS, LANES = 2, 16, 16
NW = NC * NS
EW = E // NW
K = 80
NCH = EW // K
NP = 10240
RSUB = NP // NS

R = 1024
G = NP // R

NEG = -1.0e30


def _shuf(v, idx16):
    return lax.gather(
        v, idx16[:, None],
        lax.GatherDimensionNumbers(offset_dims=(), collapsed_slice_dims=(0,),
                                   start_index_map=(0,)),
        (1,), mode=lax.GatherScatterMode.PROMISE_IN_BOUNDS)


def _edge_body1(ta_buf, d_buf, u_buf):
    lane = lax.iota(jnp.int32, 16)
    half = lane >> 3

    def f(k, c):
        a = ta_buf[k, pl.ds(64, 16)]
        dd = d_buf[k, pl.ds(0, 16)]
        e = a + dd
        w = jnp.exp(jnp.maximum(e, 0.2 * e))
        for ci in range(4):
            wb = _shuf(w, half + 2 * ci)
            u_buf[k, pl.ds(ci * 16, 16)] = wb * ta_buf[k, pl.ds(ci * 16, 16)]
        u_buf[k, pl.ds(64, 16)] = w
        return c
    return f


def _edge_body2(ta_buf, d_buf, u_buf):
    nine = jnp.full((16,), 9, jnp.int32)

    def f(k, c):
        t2 = ta_buf[k, pl.ds(32, 16)]
        sa = _shuf(t2, nine)
        e = sa + d_buf[k, pl.ds(0, 16)]
        w = jnp.exp(jnp.maximum(e, 0.2 * e))
        u_buf[k, pl.ds(0, 16)] = w * ta_buf[k, pl.ds(0, 16)]
        u_buf[k, pl.ds(16, 16)] = w * ta_buf[k, pl.ds(16, 16)]
        u_buf[k, pl.ds(32, 16)] = w * t2
        return c
    return f


def _sc_edges(ta, d, edges, ta_w, d_w, u_w, edge_body):
    mesh = plsc.VectorSubcoreMesh(core_axis_name="c", subcore_axis_name="s")

    @functools.partial(
        pl.kernel,
        out_type=jax.ShapeDtypeStruct((NC, NP, u_w), jnp.float32),
        mesh=mesh,
        scratch_types=[
            pltpu.VMEM((K,), jnp.int32),
            pltpu.VMEM((K,), jnp.int32),
            pltpu.VMEM((K,), jnp.int32),
            pltpu.VMEM((K,), jnp.int32),
            pltpu.VMEM((K,), jnp.int32),
            pltpu.VMEM((K,), jnp.int32),
            pltpu.VMEM((K, ta_w), jnp.float32),
            pltpu.VMEM((K, d_w), jnp.float32),
            pltpu.VMEM((K, ta_w), jnp.float32),
            pltpu.VMEM((K, d_w), jnp.float32),
            pltpu.VMEM((K, u_w), jnp.float32),
            pltpu.VMEM_SHARED((NP, u_w), jnp.float32),
            pltpu.SemaphoreType.DMA,
            pltpu.SemaphoreType.DMA,
            pltpu.SemaphoreType.DMA,
            pltpu.SemaphoreType.DMA,
        ],
        compiler_params=pltpu.CompilerParams(use_tc_tiling_on_sc=False),
    )
    def k(ta_hbm, d_hbm, edge_hbm, acc_hbm,
          siA, diA, dsA, siB, diB, dsB, taA, dA, taB, dB, u_buf, acc_sh,
          semA, semB, simA, simB):
        cid = lax.axis_index("c")
        sid = lax.axis_index("s")
        wid = sid * NC + cid
        r0 = sid * RSUB
        e0 = wid * EW

        def fetch_idx(j, si, di, sem):
            pltpu.async_copy(edge_hbm.at[0, pl.ds(e0 + j * K, K)], si, sem)
            pltpu.async_copy(edge_hbm.at[1, pl.ds(e0 + j * K, K)], di, sem)

        def wait_idx(si, di, sem):
            pltpu.make_async_copy(edge_hbm.at[0, pl.ds(0, K)], si, sem).wait()
            pltpu.make_async_copy(edge_hbm.at[1, pl.ds(0, K)], di, sem).wait()

        def issue(si, di, ta_buf, d_buf, sem):
            pltpu.async_copy(ta_hbm.at[si], ta_buf, sem)
            pltpu.async_copy(d_hbm.at[di], d_buf, sem)

        def drain(si, di, ta_buf, d_buf, sem):
            pltpu.make_async_copy(ta_hbm.at[si], ta_buf, sem).wait()
            pltpu.make_async_copy(d_hbm.at[di], d_buf, sem).wait()

        def copy_idx(di, ds_):
            for ci in range(K // 16):
                ds_[pl.ds(ci * 16, 16)] = di[pl.ds(ci * 16, 16)]

        fetch_idx(0, siA, diA, simA)
        fetch_idx(1, siB, diB, simB)

        def zrow(r, c):
            for ci in range(u_w // 16):
                u_buf[r, pl.ds(ci * 16, 16)] = jnp.zeros((16,), jnp.float32)
            return c
        lax.fori_loop(0, K, zrow, 0)

        def zcp(i, c):
            pltpu.sync_copy(u_buf, acc_sh.at[pl.ds(r0 + i * K, K)])
            return c
        lax.fori_loop(0, RSUB // K, zcp, 0)

        wait_idx(siA, diA, simA)
        issue(siA, diA, taA, dA, semA)
        wait_idx(siB, diB, simB)
        issue(siB, diB, taB, dB, semB)
        plsc.subcore_barrier()

        def half(j, si, di, ds_, ta_buf, d_buf, sem, sim):
            drain(si, di, ta_buf, d_buf, sem)
            copy_idx(di, ds_)
            nj = j + 2

            @pl.when(nj < NCH)
            def _():
                fetch_idx(nj, si, di, sim)

            body = edge_body(ta_buf, d_buf, u_buf)
            plsc.parallel_loop(0, K, 1, unroll=2)(lambda kk: body(kk, 0))
            pltpu.sync_copy(u_buf, acc_sh.at[ds_], add=True)

            @pl.when(nj < NCH)
            def _():
                wait_idx(si, di, sim)
                issue(si, di, ta_buf, d_buf, sem)

        def pair(p, carry):
            j = 2 * p
            half(j, siA, diA, dsA, taA, dA, semA, simA)
            half(j + 1, siB, diB, dsB, taB, dB, semB, simB)
            return carry

        lax.fori_loop(0, (NCH - 1) // 2, pair, 0)
        half(NCH - 1, siA, diA, dsA, taA, dA, semA, simA)
        plsc.subcore_barrier()
        pltpu.sync_copy(acc_sh.at[pl.ds(r0, RSUB)],
                        acc_hbm.at[cid, pl.ds(r0, RSUB)])

    return k(ta, d, edges)


def _tc1(x, W1, As8, Ad8, Rep8):
    def body(x_r, w1_r, as_r, ad_r, rep_r, ta_r, d_r, self_r):
        h = jnp.dot(x_r[...], w1_r[...], preferred_element_type=jnp.float32)
        sa = jnp.dot(h, as_r[...], preferred_element_type=jnp.float32)
        da = jnp.dot(h, ad_r[...], preferred_element_type=jnp.float32)
        e = sa + da
        w8 = jnp.exp(jnp.maximum(e, 0.2 * e))
        w64 = jnp.dot(w8, rep_r[...], preferred_element_type=jnp.float32)
        z8 = jnp.zeros((R, 8), jnp.float32)
        ta_r[...] = jnp.concatenate([h, sa, jnp.full((R, 8), NEG)], axis=1)
        d_r[...] = jnp.concatenate([da, z8], axis=1)
        self_r[...] = jnp.concatenate([w64 * h, w8, z8], axis=1)

    return pl.pallas_call(
        body,
        grid=(G,),
        in_specs=[pl.BlockSpec((R, 128), lambda i: (i, 0)),
                  pl.BlockSpec((128, 64), lambda i: (0, 0)),
                  pl.BlockSpec((64, 8), lambda i: (0, 0)),
                  pl.BlockSpec((64, 8), lambda i: (0, 0)),
                  pl.BlockSpec((8, 64), lambda i: (0, 0))],
        out_specs=[pl.BlockSpec((R, 80), lambda i: (i, 0)),
                   pl.BlockSpec((R, 16), lambda i: (i, 0)),
                   pl.BlockSpec((R, 80), lambda i: (i, 0))],
        out_shape=[jax.ShapeDtypeStruct((N, 80), jnp.float32),
                   jax.ShapeDtypeStruct((N, 16), jnp.float32),
                   jax.ShapeDtypeStruct((N, 80), jnp.float32)],
    )(x, W1, As8, Ad8, Rep8)


def _tc2(acc, self1, b1, W2, As2, Ad2, Rep8):
    def body(a0_r, a1_r, s_r, b1_r, w2_r, as2_r, ad2_r, rep_r, ta_r, d_r, self_r):
        acc_b = a0_r[0] + a1_r[0] + s_r[...]
        den = jnp.dot(acc_b[:, 64:72], rep_r[...],
                      preferred_element_type=jnp.float32)
        out1 = acc_b[:, :64] / den + b1_r[...]
        hh = jnp.where(out1 > 0, out1, jnp.exp(jnp.minimum(out1, 0.0)) - 1.0)
        h2 = jnp.dot(hh, w2_r[...], preferred_element_type=jnp.float32)
        sa = jnp.dot(h2, as2_r[...], preferred_element_type=jnp.float32)
        da = jnp.dot(h2, ad2_r[...], preferred_element_type=jnp.float32)
        e = sa + da
        w = jnp.exp(jnp.maximum(e, 0.2 * e))
        col = lax.broadcasted_iota(jnp.int32, (R, 8), 1)
        mid = jnp.where(col == 0, 1.0,
                        jnp.where(col == 1, sa[:, 0:1], 0.0)).astype(jnp.float32)
        hm = jnp.concatenate([h2, mid], axis=1)
        ta_r[...] = hm
        d_r[...] = da
        self_r[...] = hm * jnp.concatenate([w, w, w], axis=1)

    return pl.pallas_call(
        body,
        grid=(G,),
        in_specs=[pl.BlockSpec((1, R, 80), lambda i: (0, i, 0)),
                  pl.BlockSpec((1, R, 80), lambda i: (1, i, 0)),
                  pl.BlockSpec((R, 80), lambda i: (i, 0)),
                  pl.BlockSpec((1, 64), lambda i: (0, 0)),
                  pl.BlockSpec((64, 40), lambda i: (0, 0)),
                  pl.BlockSpec((40, 16), lambda i: (0, 0)),
                  pl.BlockSpec((40, 16), lambda i: (0, 0)),
                  pl.BlockSpec((8, 64), lambda i: (0, 0))],
        out_specs=[pl.BlockSpec((R, 48), lambda i: (i, 0)),
                   pl.BlockSpec((R, 16), lambda i: (i, 0)),
                   pl.BlockSpec((R, 48), lambda i: (i, 0))],
        out_shape=[jax.ShapeDtypeStruct((N, 48), jnp.float32),
                   jax.ShapeDtypeStruct((N, 16), jnp.float32),
                   jax.ShapeDtypeStruct((N, 48), jnp.float32)],
    )(acc, acc, self1, b1, W2, As2, Ad2, Rep8)


def _tc3(acc, self2, b2):
    def body(a0_r, a1_r, s_r, b2_r, o_r):
        acc_b = a0_r[0] + a1_r[0] + s_r[...]
        o = acc_b[:, :40] / acc_b[:, 40:41] + b2_r[...]
        m = jnp.max(o, axis=1, keepdims=True)
        lse = m + jnp.log(jnp.sum(jnp.exp(o - m), axis=1, keepdims=True))
        o_r[...] = o - lse

    return pl.pallas_call(
        body,
        grid=(G,),
        in_specs=[pl.BlockSpec((1, R, 48), lambda i: (0, i, 0)),
                  pl.BlockSpec((1, R, 48), lambda i: (1, i, 0)),
                  pl.BlockSpec((R, 48), lambda i: (i, 0)),
                  pl.BlockSpec((1, 40), lambda i: (0, 0))],
        out_specs=pl.BlockSpec((R, 40), lambda i: (i, 0)),
        out_shape=jax.ShapeDtypeStruct((N, 40), jnp.float32),
    )(acc, acc, self2, b2)


def kernel(x, edge_index, W1, att_src1, att_dst1, b1,
           W2, att_src2, att_dst2, b2):
    f32 = jnp.float32
    hd = jnp.arange(64) // 8
    sel = (hd[:, None] == jnp.arange(8)[None, :]).astype(f32)
    As8 = att_src1.reshape(64)[:, None] * sel
    Ad8 = att_dst1.reshape(64)[:, None] * sel
    Rep8 = sel.T

    ta1, d1, self1 = _tc1(x, W1, As8, Ad8, Rep8)
    acc1 = _sc_edges(ta1, d1, edge_index, 80, 16, 80, _edge_body1)

    As2 = jnp.broadcast_to(att_src2.reshape(40)[:, None], (40, 16)).astype(f32)
    Ad2 = jnp.broadcast_to(att_dst2.reshape(40)[:, None], (40, 16)).astype(f32)
    ta2, d2, self2 = _tc2(acc1, self1, b1.reshape(1, 64), W2, As2, Ad2, Rep8)

    acc2 = _sc_edges(ta2, d2, edge_index, 48, 16, 48, _edge_body2)

    return _tc3(acc2, self2, b2.reshape(1, 40))


# speedup vs baseline: 1.2214x; 1.0173x over previous
import functools

import jax
import jax.numpy as jnp
from jax import lax
from jax.experimental import pallas as pl
from jax.experimental.pallas import tpu as pltpu
from jax.experimental.pallas import tpu_sc as plsc

N = 10000
E = 320000
NC, NS, LANES = 2, 16, 16
NW = NC * NS
EW = E // NW
K = 80
NCH = EW // K
NP = 10240
RSUB = NP // NS

R = 1024
G = NP // R

NEG = -1.0e30


def _shuf(v, idx16):
    return lax.gather(
        v, idx16[:, None],
        lax.GatherDimensionNumbers(offset_dims=(), collapsed_slice_dims=(0,),
                                   start_index_map=(0,)),
        (1,), mode=lax.GatherScatterMode.PROMISE_IN_BOUNDS)


def _edge_body1(ta_buf, d_buf, u_buf):
    lane = lax.iota(jnp.int32, 16)
    half = lane >> 3

    def f(k, c):
        a = ta_buf[k, pl.ds(64, 16)]
        dd = d_buf[k, pl.ds(0, 16)]
        e = a + dd
        w = jnp.exp(jnp.maximum(e, 0.2 * e))
        for ci in range(4):
            wb = _shuf(w, half + 2 * ci)
            u_buf[k, pl.ds(ci * 16, 16)] = wb * ta_buf[k, pl.ds(ci * 16, 16)]
        u_buf[k, pl.ds(64, 16)] = w
        return c
    return f


def _edge_body2(ta_buf, d_buf, u_buf):
    nine = jnp.full((16,), 9, jnp.int32)

    def f(k, c):
        t2 = ta_buf[k, pl.ds(32, 16)]
        sa = _shuf(t2, nine)
        e = sa + d_buf[k, pl.ds(0, 16)]
        w = jnp.exp(jnp.maximum(e, 0.2 * e))
        u_buf[k, pl.ds(0, 16)] = w * ta_buf[k, pl.ds(0, 16)]
        u_buf[k, pl.ds(16, 16)] = w * ta_buf[k, pl.ds(16, 16)]
        u_buf[k, pl.ds(32, 16)] = w * t2
        return c
    return f


def _sc_edges(ta, d, edges, ta_w, d_w, u_w, edge_body):
    mesh = plsc.VectorSubcoreMesh(core_axis_name="c", subcore_axis_name="s")

    @functools.partial(
        pl.kernel,
        out_type=jax.ShapeDtypeStruct((NC, NP, u_w), jnp.float32),
        mesh=mesh,
        scratch_types=[
            pltpu.VMEM((K,), jnp.int32),
            pltpu.VMEM((K,), jnp.int32),
            pltpu.VMEM((K,), jnp.int32),
            pltpu.VMEM((K,), jnp.int32),
            pltpu.VMEM((K,), jnp.int32),
            pltpu.VMEM((K,), jnp.int32),
            pltpu.VMEM((K, ta_w), jnp.float32),
            pltpu.VMEM((K, d_w), jnp.float32),
            pltpu.VMEM((K, ta_w), jnp.float32),
            pltpu.VMEM((K, d_w), jnp.float32),
            pltpu.VMEM((K, u_w), jnp.float32),
            pltpu.VMEM((K, u_w), jnp.float32),
            pltpu.VMEM_SHARED((NP, u_w), jnp.float32),
            pltpu.SemaphoreType.DMA,
            pltpu.SemaphoreType.DMA,
            pltpu.SemaphoreType.DMA,
            pltpu.SemaphoreType.DMA,
            pltpu.SemaphoreType.DMA,
            pltpu.SemaphoreType.DMA,
        ],
        compiler_params=pltpu.CompilerParams(use_tc_tiling_on_sc=False),
    )
    def k(ta_hbm, d_hbm, edge_hbm, acc_hbm,
          siA, diA, dsA, siB, diB, dsB, taA, dA, taB, dB, uA, uB, acc_sh,
          semA, semB, simA, simB, ssemA, ssemB):
        cid = lax.axis_index("c")
        sid = lax.axis_index("s")
        wid = sid * NC + cid
        r0 = sid * RSUB
        e0 = wid * EW

        def fetch_idx(j, si, di, sem):
            pltpu.async_copy(edge_hbm.at[0, pl.ds(e0 + j * K, K)], si, sem)
            pltpu.async_copy(edge_hbm.at[1, pl.ds(e0 + j * K, K)], di, sem)

        def wait_idx(si, di, sem):
            pltpu.make_async_copy(edge_hbm.at[0, pl.ds(0, K)], si, sem).wait()
            pltpu.make_async_copy(edge_hbm.at[1, pl.ds(0, K)], di, sem).wait()

        def issue(si, di, ta_buf, d_buf, sem):
            pltpu.async_copy(ta_hbm.at[si], ta_buf, sem)
            pltpu.async_copy(d_hbm.at[di], d_buf, sem)

        def drain(si, di, ta_buf, d_buf, sem):
            pltpu.make_async_copy(ta_hbm.at[si], ta_buf, sem).wait()
            pltpu.make_async_copy(d_hbm.at[di], d_buf, sem).wait()

        def copy_idx(di, ds_):
            for ci in range(K // 16):
                ds_[pl.ds(ci * 16, 16)] = di[pl.ds(ci * 16, 16)]

        fetch_idx(0, siA, diA, simA)
        fetch_idx(1, siB, diB, simB)

        def zrow(r, c):
            for ci in range(u_w // 16):
                uA[r, pl.ds(ci * 16, 16)] = jnp.zeros((16,), jnp.float32)
            return c
        lax.fori_loop(0, K, zrow, 0)

        def zcp(i, c):
            pltpu.sync_copy(uA, acc_sh.at[pl.ds(r0 + i * K, K)])
            return c
        lax.fori_loop(0, RSUB // K, zcp, 0)

        wait_idx(siA, diA, simA)
        issue(siA, diA, taA, dA, semA)
        wait_idx(siB, diB, simB)
        issue(siB, diB, taB, dB, semB)
        plsc.subcore_barrier()

        def half(j, si, di, ds_, ta_buf, d_buf, u_buf, sem, sim, ssem):
            drain(si, di, ta_buf, d_buf, sem)

            @pl.when(j >= 2)
            def _():
                pltpu.make_async_copy(u_buf, acc_sh.at[ds_], ssem).wait()

            copy_idx(di, ds_)
            nj = j + 2

            @pl.when(nj < NCH)
            def _():
                fetch_idx(nj, si, di, sim)

            body = edge_body(ta_buf, d_buf, u_buf)
            plsc.parallel_loop(0, K, 1, unroll=2)(lambda kk: body(kk, 0))
            pltpu.async_copy(u_buf, acc_sh.at[ds_], ssem, add=True)

            @pl.when(nj < NCH)
            def _():
                wait_idx(si, di, sim)
                issue(si, di, ta_buf, d_buf, sem)

        def pair(p, carry):
            j = 2 * p
            half(j, siA, diA, dsA, taA, dA, uA, semA, simA, ssemA)
            half(j + 1, siB, diB, dsB, taB, dB, uB, semB, simB, ssemB)
            return carry

        lax.fori_loop(0, (NCH - 1) // 2, pair, 0)
        half(NCH - 1, siA, diA, dsA, taA, dA, uA, semA, simA, ssemA)
        pltpu.make_async_copy(uA, acc_sh.at[dsA], ssemA).wait()
        pltpu.make_async_copy(uB, acc_sh.at[dsB], ssemB).wait()
        plsc.subcore_barrier()
        pltpu.sync_copy(acc_sh.at[pl.ds(r0, RSUB)],
                        acc_hbm.at[cid, pl.ds(r0, RSUB)])

    return k(ta, d, edges)


def _tc1(x, W1, As8, Ad8, Rep8):
    def body(x_r, w1_r, as_r, ad_r, rep_r, ta_r, d_r, self_r):
        h = jnp.dot(x_r[...], w1_r[...], preferred_element_type=jnp.float32)
        sa = jnp.dot(h, as_r[...], preferred_element_type=jnp.float32)
        da = jnp.dot(h, ad_r[...], preferred_element_type=jnp.float32)
        e = sa + da
        w8 = jnp.exp(jnp.maximum(e, 0.2 * e))
        w64 = jnp.dot(w8, rep_r[...], preferred_element_type=jnp.float32)
        z8 = jnp.zeros((R, 8), jnp.float32)
        ta_r[...] = jnp.concatenate([h, sa, jnp.full((R, 8), NEG)], axis=1)
        d_r[...] = jnp.concatenate([da, z8], axis=1)
        self_r[...] = jnp.concatenate([w64 * h, w8, z8], axis=1)

    return pl.pallas_call(
        body,
        grid=(G,),
        in_specs=[pl.BlockSpec((R, 128), lambda i: (i, 0)),
                  pl.BlockSpec((128, 64), lambda i: (0, 0)),
                  pl.BlockSpec((64, 8), lambda i: (0, 0)),
                  pl.BlockSpec((64, 8), lambda i: (0, 0)),
                  pl.BlockSpec((8, 64), lambda i: (0, 0))],
        out_specs=[pl.BlockSpec((R, 80), lambda i: (i, 0)),
                   pl.BlockSpec((R, 16), lambda i: (i, 0)),
                   pl.BlockSpec((R, 80), lambda i: (i, 0))],
        out_shape=[jax.ShapeDtypeStruct((N, 80), jnp.float32),
                   jax.ShapeDtypeStruct((N, 16), jnp.float32),
                   jax.ShapeDtypeStruct((N, 80), jnp.float32)],
    )(x, W1, As8, Ad8, Rep8)


def _tc2(acc, self1, b1, W2, As2, Ad2, Rep8):
    def body(a0_r, a1_r, s_r, b1_r, w2_r, as2_r, ad2_r, rep_r, ta_r, d_r, self_r):
        acc_b = a0_r[0] + a1_r[0] + s_r[...]
        den = jnp.dot(acc_b[:, 64:72], rep_r[...],
                      preferred_element_type=jnp.float32)
        out1 = acc_b[:, :64] / den + b1_r[...]
        hh = jnp.where(out1 > 0, out1, jnp.exp(jnp.minimum(out1, 0.0)) - 1.0)
        h2 = jnp.dot(hh, w2_r[...], preferred_element_type=jnp.float32)
        sa = jnp.dot(h2, as2_r[...], preferred_element_type=jnp.float32)
        da = jnp.dot(h2, ad2_r[...], preferred_element_type=jnp.float32)
        e = sa + da
        w = jnp.exp(jnp.maximum(e, 0.2 * e))
        col = lax.broadcasted_iota(jnp.int32, (R, 8), 1)
        mid = jnp.where(col == 0, 1.0,
                        jnp.where(col == 1, sa[:, 0:1], 0.0)).astype(jnp.float32)
        hm = jnp.concatenate([h2, mid], axis=1)
        ta_r[...] = hm
        d_r[...] = da
        self_r[...] = hm * jnp.concatenate([w, w, w], axis=1)

    return pl.pallas_call(
        body,
        grid=(G,),
        in_specs=[pl.BlockSpec((1, R, 80), lambda i: (0, i, 0)),
                  pl.BlockSpec((1, R, 80), lambda i: (1, i, 0)),
                  pl.BlockSpec((R, 80), lambda i: (i, 0)),
                  pl.BlockSpec((1, 64), lambda i: (0, 0)),
                  pl.BlockSpec((64, 40), lambda i: (0, 0)),
                  pl.BlockSpec((40, 16), lambda i: (0, 0)),
                  pl.BlockSpec((40, 16), lambda i: (0, 0)),
                  pl.BlockSpec((8, 64), lambda i: (0, 0))],
        out_specs=[pl.BlockSpec((R, 48), lambda i: (i, 0)),
                   pl.BlockSpec((R, 16), lambda i: (i, 0)),
                   pl.BlockSpec((R, 48), lambda i: (i, 0))],
        out_shape=[jax.ShapeDtypeStruct((N, 48), jnp.float32),
                   jax.ShapeDtypeStruct((N, 16), jnp.float32),
                   jax.ShapeDtypeStruct((N, 48), jnp.float32)],
    )(acc, acc, self1, b1, W2, As2, Ad2, Rep8)


def _tc3(acc, self2, b2):
    def body(a0_r, a1_r, s_r, b2_r, o_r):
        acc_b = a0_r[0] + a1_r[0] + s_r[...]
        o = acc_b[:, :40] / acc_b[:, 40:41] + b2_r[...]
        m = jnp.max(o, axis=1, keepdims=True)
        lse = m + jnp.log(jnp.sum(jnp.exp(o - m), axis=1, keepdims=True))
        o_r[...] = o - lse

    return pl.pallas_call(
        body,
        grid=(G,),
        in_specs=[pl.BlockSpec((1, R, 48), lambda i: (0, i, 0)),
                  pl.BlockSpec((1, R, 48), lambda i: (1, i, 0)),
                  pl.BlockSpec((R, 48), lambda i: (i, 0)),
                  pl.BlockSpec((1, 40), lambda i: (0, 0))],
        out_specs=pl.BlockSpec((R, 40), lambda i: (i, 0)),
        out_shape=jax.ShapeDtypeStruct((N, 40), jnp.float32),
    )(acc, acc, self2, b2)


def kernel(x, edge_index, W1, att_src1, att_dst1, b1,
           W2, att_src2, att_dst2, b2):
    f32 = jnp.float32
    hd = jnp.arange(64) // 8
    sel = (hd[:, None] == jnp.arange(8)[None, :]).astype(f32)
    As8 = att_src1.reshape(64)[:, None] * sel
    Ad8 = att_dst1.reshape(64)[:, None] * sel
    Rep8 = sel.T

    ta1, d1, self1 = _tc1(x, W1, As8, Ad8, Rep8)
    acc1 = _sc_edges(ta1, d1, edge_index, 80, 16, 80, _edge_body1)

    As2 = jnp.broadcast_to(att_src2.reshape(40)[:, None], (40, 16)).astype(f32)
    Ad2 = jnp.broadcast_to(att_dst2.reshape(40)[:, None], (40, 16)).astype(f32)
    ta2, d2, self2 = _tc2(acc1, self1, b1.reshape(1, 64), W2, As2, Ad2, Rep8)

    acc2 = _sc_edges(ta2, d2, edge_index, 48, 16, 48, _edge_body2)

    return _tc3(acc2, self2, b2.reshape(1, 40))


# speedup vs baseline: 1.2469x; 1.0209x over previous
import functools

import jax
import jax.numpy as jnp
from jax import lax
from jax.experimental import pallas as pl
from jax.experimental.pallas import tpu as pltpu
from jax.experimental.pallas import tpu_sc as plsc

N = 10000
E = 320000
NC, NS, LANES = 2, 16, 16
NW = NC * NS
EW = E // NW
K = 80
NCH = EW // K
NP = 10240
RSUB = NP // NS

R = 1024
G = NP // R

NEG = -1.0e30
AW = 128


def _shuf(v, idx16):
    return lax.gather(
        v, idx16[:, None],
        lax.GatherDimensionNumbers(offset_dims=(), collapsed_slice_dims=(0,),
                                   start_index_map=(0,)),
        (1,), mode=lax.GatherScatterMode.PROMISE_IN_BOUNDS)


def _edge_body1(ta_buf, d_buf, u_buf):
    lane = lax.iota(jnp.int32, 16)
    half = lane >> 3

    def f(k, c):
        a = ta_buf[k, pl.ds(64, 16)]
        dd = d_buf[k, pl.ds(0, 16)]
        e = a + dd
        w = jnp.exp(jnp.maximum(e, 0.2 * e))
        for ci in range(4):
            wb = _shuf(w, half + 2 * ci)
            u_buf[k, pl.ds(ci * 16, 16)] = wb * ta_buf[k, pl.ds(ci * 16, 16)]
        u_buf[k, pl.ds(64, 16)] = w
        return c
    return f


def _edge_body2(ta_buf, d_buf, u_buf):
    nine = jnp.full((16,), 9, jnp.int32)

    def f(k, c):
        t2 = ta_buf[k, pl.ds(32, 16)]
        sa = _shuf(t2, nine)
        e = sa + d_buf[k, pl.ds(0, 16)]
        w = jnp.exp(jnp.maximum(e, 0.2 * e))
        u_buf[k, pl.ds(0, 16)] = w * ta_buf[k, pl.ds(0, 16)]
        u_buf[k, pl.ds(16, 16)] = w * ta_buf[k, pl.ds(16, 16)]
        u_buf[k, pl.ds(32, 16)] = w * t2
        return c
    return f


def _sc_edges(ta, d, edges, ta_w, d_w, u_w, edge_body):
    mesh = plsc.VectorSubcoreMesh(core_axis_name="c", subcore_axis_name="s")

    @functools.partial(
        pl.kernel,
        out_type=jax.ShapeDtypeStruct((NC, NP, AW), jnp.float32),
        mesh=mesh,
        scratch_types=[
            pltpu.VMEM((K,), jnp.int32),
            pltpu.VMEM((K,), jnp.int32),
            pltpu.VMEM((K,), jnp.int32),
            pltpu.VMEM((K,), jnp.int32),
            pltpu.VMEM((K,), jnp.int32),
            pltpu.VMEM((K,), jnp.int32),
            pltpu.VMEM((K, ta_w), jnp.float32),
            pltpu.VMEM((K, d_w), jnp.float32),
            pltpu.VMEM((K, ta_w), jnp.float32),
            pltpu.VMEM((K, d_w), jnp.float32),
            pltpu.VMEM((K, AW), jnp.float32),
            pltpu.VMEM((K, AW), jnp.float32),
            pltpu.VMEM_SHARED((NP, AW), jnp.float32),
            pltpu.SemaphoreType.DMA,
            pltpu.SemaphoreType.DMA,
            pltpu.SemaphoreType.DMA,
            pltpu.SemaphoreType.DMA,
            pltpu.SemaphoreType.DMA,
            pltpu.SemaphoreType.DMA,
        ],
        compiler_params=pltpu.CompilerParams(use_tc_tiling_on_sc=False),
    )
    def k(ta_hbm, d_hbm, edge_hbm, acc_hbm,
          siA, diA, dsA, siB, diB, dsB, taA, dA, taB, dB, uA, uB, acc_sh,
          semA, semB, simA, simB, ssemA, ssemB):
        cid = lax.axis_index("c")
        sid = lax.axis_index("s")
        wid = sid * NC + cid
        r0 = sid * RSUB
        e0 = wid * EW

        def fetch_idx(j, si, di, sem):
            pltpu.async_copy(edge_hbm.at[0, pl.ds(e0 + j * K, K)], si, sem)
            pltpu.async_copy(edge_hbm.at[1, pl.ds(e0 + j * K, K)], di, sem)

        def wait_idx(si, di, sem):
            pltpu.make_async_copy(edge_hbm.at[0, pl.ds(0, K)], si, sem).wait()
            pltpu.make_async_copy(edge_hbm.at[1, pl.ds(0, K)], di, sem).wait()

        def issue(si, di, ta_buf, d_buf, sem):
            pltpu.async_copy(ta_hbm.at[si], ta_buf, sem)
            pltpu.async_copy(d_hbm.at[di], d_buf, sem)

        def drain(si, di, ta_buf, d_buf, sem):
            pltpu.make_async_copy(ta_hbm.at[si], ta_buf, sem).wait()
            pltpu.make_async_copy(d_hbm.at[di], d_buf, sem).wait()

        def copy_idx(di, ds_):
            for ci in range(K // 16):
                ds_[pl.ds(ci * 16, 16)] = di[pl.ds(ci * 16, 16)]

        fetch_idx(0, siA, diA, simA)
        fetch_idx(1, siB, diB, simB)

        def zrow(r, c):
            for ci in range(AW // 16):
                uA[r, pl.ds(ci * 16, 16)] = jnp.zeros((16,), jnp.float32)
                uB[r, pl.ds(ci * 16, 16)] = jnp.zeros((16,), jnp.float32)
            return c
        lax.fori_loop(0, K, zrow, 0)

        def zcp(i, c):
            pltpu.sync_copy(uA, acc_sh.at[pl.ds(r0 + i * K, K)])
            return c
        lax.fori_loop(0, RSUB // K, zcp, 0)

        wait_idx(siA, diA, simA)
        issue(siA, diA, taA, dA, semA)
        wait_idx(siB, diB, simB)
        issue(siB, diB, taB, dB, semB)
        plsc.subcore_barrier()

        def half(j, si, di, ds_, ta_buf, d_buf, u_buf, sem, sim, ssem):
            drain(si, di, ta_buf, d_buf, sem)

            @pl.when(j >= 2)
            def _():
                pltpu.make_async_copy(u_buf, acc_sh.at[ds_], ssem).wait()

            copy_idx(di, ds_)
            nj = j + 2

            @pl.when(nj < NCH)
            def _():
                fetch_idx(nj, si, di, sim)

            body = edge_body(ta_buf, d_buf, u_buf)
            plsc.parallel_loop(0, K, 1, unroll=2)(lambda kk: body(kk, 0))
            pltpu.async_copy(u_buf, acc_sh.at[ds_], ssem, add=True)

            @pl.when(nj < NCH)
            def _():
                wait_idx(si, di, sim)
                issue(si, di, ta_buf, d_buf, sem)

        def pair(p, carry):
            j = 2 * p
            half(j, siA, diA, dsA, taA, dA, uA, semA, simA, ssemA)
            half(j + 1, siB, diB, dsB, taB, dB, uB, semB, simB, ssemB)
            return carry

        lax.fori_loop(0, (NCH - 1) // 2, pair, 0)
        half(NCH - 1, siA, diA, dsA, taA, dA, uA, semA, simA, ssemA)
        pltpu.make_async_copy(uA, acc_sh.at[dsA], ssemA).wait()
        pltpu.make_async_copy(uB, acc_sh.at[dsB], ssemB).wait()
        plsc.subcore_barrier()
        pltpu.sync_copy(acc_sh.at[pl.ds(r0, RSUB)],
                        acc_hbm.at[cid, pl.ds(r0, RSUB)])

    return k(ta, d, edges)


def _tc1(x, W1, As8, Ad8, Rep8):
    def body(x_r, w1_r, as_r, ad_r, rep_r, ta_r, d_r, self_r):
        h = jnp.dot(x_r[...], w1_r[...], preferred_element_type=jnp.float32)
        sa = jnp.dot(h, as_r[...], preferred_element_type=jnp.float32)
        da = jnp.dot(h, ad_r[...], preferred_element_type=jnp.float32)
        e = sa + da
        w8 = jnp.exp(jnp.maximum(e, 0.2 * e))
        w64 = jnp.dot(w8, rep_r[...], preferred_element_type=jnp.float32)
        z8 = jnp.zeros((R, 8), jnp.float32)
        ta_r[...] = jnp.concatenate([h, sa, jnp.full((R, 8), NEG)], axis=1)
        d_r[...] = jnp.concatenate([da, z8], axis=1)
        self_r[...] = jnp.concatenate([w64 * h, w8, z8], axis=1)

    return pl.pallas_call(
        body,
        grid=(G,),
        in_specs=[pl.BlockSpec((R, 128), lambda i: (i, 0)),
                  pl.BlockSpec((128, 64), lambda i: (0, 0)),
                  pl.BlockSpec((64, 8), lambda i: (0, 0)),
                  pl.BlockSpec((64, 8), lambda i: (0, 0)),
                  pl.BlockSpec((8, 64), lambda i: (0, 0))],
        out_specs=[pl.BlockSpec((R, 80), lambda i: (i, 0)),
                   pl.BlockSpec((R, 16), lambda i: (i, 0)),
                   pl.BlockSpec((R, 80), lambda i: (i, 0))],
        out_shape=[jax.ShapeDtypeStruct((N, 80), jnp.float32),
                   jax.ShapeDtypeStruct((N, 16), jnp.float32),
                   jax.ShapeDtypeStruct((N, 80), jnp.float32)],
    )(x, W1, As8, Ad8, Rep8)


def _tc2(acc, self1, b1, W2, As2, Ad2, Rep8):
    def body(a0_r, a1_r, s_r, b1_r, w2_r, as2_r, ad2_r, rep_r, ta_r, d_r, self_r):
        acc_b = a0_r[0] + a1_r[0]
        s = s_r[...]
        den = jnp.dot(acc_b[:, 64:72] + s[:, 64:72], rep_r[...],
                      preferred_element_type=jnp.float32)
        out1 = (acc_b[:, :64] + s[:, :64]) / den + b1_r[...]
        hh = jnp.where(out1 > 0, out1, jnp.exp(jnp.minimum(out1, 0.0)) - 1.0)
        h2 = jnp.dot(hh, w2_r[...], preferred_element_type=jnp.float32)
        sa = jnp.dot(h2, as2_r[...], preferred_element_type=jnp.float32)
        da = jnp.dot(h2, ad2_r[...], preferred_element_type=jnp.float32)
        e = sa + da
        w = jnp.exp(jnp.maximum(e, 0.2 * e))
        col = lax.broadcasted_iota(jnp.int32, (R, 8), 1)
        mid = jnp.where(col == 0, 1.0,
                        jnp.where(col == 1, sa[:, 0:1], 0.0)).astype(jnp.float32)
        hm = jnp.concatenate([h2, mid], axis=1)
        ta_r[...] = hm
        d_r[...] = da
        self_r[...] = hm * jnp.concatenate([w, w, w], axis=1)

    return pl.pallas_call(
        body,
        grid=(G,),
        in_specs=[pl.BlockSpec((1, R, AW), lambda i: (0, i, 0)),
                  pl.BlockSpec((1, R, AW), lambda i: (1, i, 0)),
                  pl.BlockSpec((R, 80), lambda i: (i, 0)),
                  pl.BlockSpec((1, 64), lambda i: (0, 0)),
                  pl.BlockSpec((64, 40), lambda i: (0, 0)),
                  pl.BlockSpec((40, 16), lambda i: (0, 0)),
                  pl.BlockSpec((40, 16), lambda i: (0, 0)),
                  pl.BlockSpec((8, 64), lambda i: (0, 0))],
        out_specs=[pl.BlockSpec((R, 48), lambda i: (i, 0)),
                   pl.BlockSpec((R, 16), lambda i: (i, 0)),
                   pl.BlockSpec((R, 48), lambda i: (i, 0))],
        out_shape=[jax.ShapeDtypeStruct((N, 48), jnp.float32),
                   jax.ShapeDtypeStruct((N, 16), jnp.float32),
                   jax.ShapeDtypeStruct((N, 48), jnp.float32)],
    )(acc, acc, self1, b1, W2, As2, Ad2, Rep8)


def _tc3(acc, self2, b2):
    def body(a0_r, a1_r, s_r, b2_r, o_r):
        acc_b = a0_r[0] + a1_r[0]
        s = s_r[...]
        o = (acc_b[:, :40] + s[:, :40]) / (acc_b[:, 40:41] + s[:, 40:41]) \
            + b2_r[...]
        m = jnp.max(o, axis=1, keepdims=True)
        lse = m + jnp.log(jnp.sum(jnp.exp(o - m), axis=1, keepdims=True))
        o_r[...] = o - lse

    return pl.pallas_call(
        body,
        grid=(G,),
        in_specs=[pl.BlockSpec((1, R, AW), lambda i: (0, i, 0)),
                  pl.BlockSpec((1, R, AW), lambda i: (1, i, 0)),
                  pl.BlockSpec((R, 48), lambda i: (i, 0)),
                  pl.BlockSpec((1, 40), lambda i: (0, 0))],
        out_specs=pl.BlockSpec((R, 40), lambda i: (i, 0)),
        out_shape=jax.ShapeDtypeStruct((N, 40), jnp.float32),
    )(acc, acc, self2, b2)


def kernel(x, edge_index, W1, att_src1, att_dst1, b1,
           W2, att_src2, att_dst2, b2):
    f32 = jnp.float32
    hd = jnp.arange(64) // 8
    sel = (hd[:, None] == jnp.arange(8)[None, :]).astype(f32)
    As8 = att_src1.reshape(64)[:, None] * sel
    Ad8 = att_dst1.reshape(64)[:, None] * sel
    Rep8 = sel.T

    ta1, d1, self1 = _tc1(x, W1, As8, Ad8, Rep8)
    acc1 = _sc_edges(ta1, d1, edge_index, 80, 16, 80, _edge_body1)

    As2 = jnp.broadcast_to(att_src2.reshape(40)[:, None], (40, 16)).astype(f32)
    Ad2 = jnp.broadcast_to(att_dst2.reshape(40)[:, None], (40, 16)).astype(f32)
    ta2, d2, self2 = _tc2(acc1, self1, b1.reshape(1, 64), W2, As2, Ad2, Rep8)

    acc2 = _sc_edges(ta2, d2, edge_index, 48, 16, 48, _edge_body2)

    return _tc3(acc2, self2, b2.reshape(1, 40))
